# Initial kernel scaffold; baseline (speedup 1.0000x reference)
import jax
import jax.numpy as jnp
from jax.experimental import pallas as pl


def kernel(protein_x, ligand_x, action, pr_in_Wr, pr_in_Ws, pr_in_b, pr_out_Wr, pr_out_Ws, pr_out_b, lg_in_Wr, lg_in_Ws, lg_in_b, lg_out_Wr, lg_out_Ws, lg_out_b, pin_W, pin_b, ph_W, ph_b, po_W, po_b, protein_edge_index, ligand_edge_index):
    raise NotImplementedError("write your pallas kernel here")


# speedup vs baseline: 9.0090x; 9.0090x over previous
import functools

import jax
import jax.numpy as jnp
from jax import lax
from jax.experimental import pallas as pl
from jax.experimental.pallas import tpu as pltpu
from jax.experimental.pallas import tpu_sc as plsc

N = 10000
E = 320000
D = 128
A = 32

NPAD = 10240
ROWS_PT = NPAD // 16
CHUNK = 128
NCHUNK = 157
EPT = NCHUNK * CHUNK
EPAD = 16 * EPT


def _sc_round_body(table, src_h, dst_h, zeros_h, acc_out,
                   src_v, dst_v, msg_v, row_v, acc_s, sem):
    cid = lax.axis_index("c")
    sid = lax.axis_index("s")
    wid = cid * 16 + sid
    lrow = sid * ROWS_PT

    pltpu.sync_copy(src_h.at[wid], src_v)
    pltpu.sync_copy(dst_h.at[wid], dst_v)
    pltpu.sync_copy(zeros_h, row_v)
    pltpu.sync_copy(row_v, acc_s.at[pl.ds(lrow, ROWS_PT)])
    plsc.subcore_barrier()

    def chunk(j, carry):
        pltpu.async_copy(table.at[src_v.at[j]], msg_v, sem).wait()
        pltpu.sync_copy(msg_v, acc_s.at[dst_v.at[j]], add=True)
        return carry

    lax.fori_loop(0, NCHUNK, chunk, 0)
    plsc.subcore_barrier()

    pltpu.sync_copy(acc_s.at[pl.ds(lrow, ROWS_PT)], row_v)
    pltpu.sync_copy(row_v, acc_out.at[pl.ds(cid * NPAD + lrow, ROWS_PT)])


@jax.jit
def _sc_round(table, src_all, dst_all, zeros_rows):
    mesh = plsc.VectorSubcoreMesh(core_axis_name="c", subcore_axis_name="s")
    return pl.kernel(
        _sc_round_body,
        out_type=jax.ShapeDtypeStruct((2 * NPAD, 16), jnp.float32),
        mesh=mesh,
        scratch_types=[
            pltpu.VMEM((NCHUNK, CHUNK), jnp.int32),
            pltpu.VMEM((NCHUNK, CHUNK), jnp.int32),
            pltpu.VMEM((CHUNK, 16), jnp.float32),
            pltpu.VMEM((ROWS_PT, 16), jnp.float32),
            pltpu.VMEM_SHARED((NPAD, 16), jnp.float32),
            pltpu.SemaphoreType.DMA,
        ],
        compiler_params=pltpu.CompilerParams(use_tc_tiling_on_sc=False),
    )(table, src_all, dst_all, zeros_rows)


def _pre_body(px, lx, wrp, wsp, bp, wrl, wsl, bl, m1p, d1p, m1l, d1l):
    xp = px[...]
    xl = lx[...]
    m1p[...] = jnp.dot(xp, wrp[...], preferred_element_type=jnp.float32)
    d1p[...] = jnp.dot(xp, wsp[...], preferred_element_type=jnp.float32) + bp[...]
    m1l[...] = jnp.dot(xl, wrl[...], preferred_element_type=jnp.float32)
    d1l[...] = jnp.dot(xl, wsl[...], preferred_element_type=jnp.float32) + bl[...]


def _mid_body(a, d, o):
    o[...] = jnp.maximum(a[...] + d[...], 0.0)


def _head_body(a2p, h1p, a2l, h1l, act,
               wrp2, wsp2, bp2, wrl2, wsl2, bl2,
               pin_w, pin_b, ph_w, ph_b, po_w, po_b, out):
    p2 = (jnp.dot(a2p[...], wrp2[...], preferred_element_type=jnp.float32)
          + jnp.dot(h1p[...], wsp2[...], preferred_element_type=jnp.float32)
          + bp2[...])
    l2 = (jnp.dot(a2l[...], wrl2[...], preferred_element_type=jnp.float32)
          + jnp.dot(h1l[...], wsl2[...], preferred_element_type=jnp.float32)
          + bl2[...])
    mol = jnp.concatenate([p2, l2], axis=1)
    fp = jnp.maximum(
        jnp.dot(mol, pin_w[...], preferred_element_type=jnp.float32) + pin_b[...],
        0.0)
    pol = (jnp.dot(jnp.concatenate([fp, act[...]], axis=1), ph_w[...],
                   preferred_element_type=jnp.float32) + ph_b[...])
    out[...] = (jnp.dot(jnp.maximum(pol, 0.0), po_w[...],
                        preferred_element_type=jnp.float32) + po_b[...])


def _row_spec(block, width):
    return pl.BlockSpec((block, width), lambda i: (i, 0))


def _full_spec():
    return pl.BlockSpec()


def _pad_edges(src, dst, src_off, pad_src):
    src = jnp.concatenate(
        [src + src_off, jnp.full((EPAD - E,), pad_src, dtype=jnp.int32)])
    dst = jnp.concatenate([dst, jnp.zeros((EPAD - E,), dtype=jnp.int32)])
    return src, dst


def kernel(protein_x, ligand_x, action,
           pr_in_Wr, pr_in_Ws, pr_in_b, pr_out_Wr, pr_out_Ws, pr_out_b,
           lg_in_Wr, lg_in_Ws, lg_in_b, lg_out_Wr, lg_out_Ws, lg_out_b,
           pin_W, pin_b, ph_W, ph_b, po_W, po_b,
           protein_edge_index, ligand_edge_index):
    f32 = jnp.float32

    sp, dp = _pad_edges(protein_edge_index[0], protein_edge_index[1], 0, N)
    sl, dl = _pad_edges(ligand_edge_index[0], ligand_edge_index[1], NPAD, NPAD + N)
    src_all = jnp.concatenate([sp, sl]).reshape(32, NCHUNK, CHUNK)
    dst_all = jnp.concatenate([dp, dl]).reshape(32, NCHUNK, CHUNK)
    zeros_rows = jnp.zeros((ROWS_PT, 16), f32)

    blk = 2000
    grid = (N // blk,)
    m1p, d1p, m1l, d1l = pl.pallas_call(
        _pre_body,
        grid=grid,
        in_specs=[_row_spec(blk, D), _row_spec(blk, D)] + [_full_spec()] * 6,
        out_specs=[_row_spec(blk, 16)] * 4,
        out_shape=[jax.ShapeDtypeStruct((N, 16), f32)] * 4,
    )(protein_x, ligand_x,
      pr_in_Wr.T, pr_in_Ws.T, pr_in_b.reshape(1, 16),
      lg_in_Wr.T, lg_in_Ws.T, lg_in_b.reshape(1, 16))

    table1 = jnp.zeros((2 * NPAD, 16), f32)
    table1 = lax.dynamic_update_slice(table1, m1p, (0, 0))
    table1 = lax.dynamic_update_slice(table1, m1l, (NPAD, 0))
    d1 = jnp.zeros((2 * NPAD, 16), f32)
    d1 = lax.dynamic_update_slice(d1, d1p, (0, 0))
    d1 = lax.dynamic_update_slice(d1, d1l, (NPAD, 0))

    acc1 = _sc_round(table1, src_all, dst_all, zeros_rows)

    mblk = 2048
    table2 = pl.pallas_call(
        _mid_body,
        grid=(2 * NPAD // mblk,),
        in_specs=[_row_spec(mblk, 16)] * 2,
        out_specs=_row_spec(mblk, 16),
        out_shape=jax.ShapeDtypeStruct((2 * NPAD, 16), f32),
    )(acc1, d1)

    acc2 = _sc_round(table2, src_all, dst_all, zeros_rows)

    h1p = lax.dynamic_slice(table2, (0, 0), (N, 16))
    h1l = lax.dynamic_slice(table2, (NPAD, 0), (N, 16))
    a2p = lax.dynamic_slice(acc2, (0, 0), (N, 16))
    a2l = lax.dynamic_slice(acc2, (NPAD, 0), (N, 16))

    out = pl.pallas_call(
        _head_body,
        grid=grid,
        in_specs=[_row_spec(blk, 16)] * 4 + [_row_spec(blk, A)]
                 + [_full_spec()] * 12,
        out_specs=_row_spec(blk, 1),
        out_shape=jax.ShapeDtypeStruct((N, 1), f32),
    )(a2p, h1p, a2l, h1l, action,
      pr_out_Wr.T, pr_out_Ws.T, pr_out_b.reshape(1, 50),
      lg_out_Wr.T, lg_out_Ws.T, lg_out_b.reshape(1, 50),
      pin_W.T, pin_b.reshape(1, 60), ph_W.T, ph_b.reshape(1, 10),
      po_W.T, po_b.reshape(1, 1))
    return out


# speedup vs baseline: 15.5770x; 1.7290x over previous
import functools

import jax
import jax.numpy as jnp
from jax import lax
from jax.experimental import pallas as pl
from jax.experimental.pallas import tpu as pltpu
from jax.experimental.pallas import tpu_sc as plsc

N = 10000
E = 320000
D = 128
A = 32

NPAD = 10240
ROWS_PT = NPAD // 16
CHUNK = 128
NCHUNK = 157
EPT = NCHUNK * CHUNK
EPAD = 16 * EPT


DEPTH = 6
NBUF = 8


def _sc_round_body(table, src_h, dst_h, zeros_h, acc_out,
                   src_v, dst_v, msg_v, row_v, acc_s, sem_g, sem_s):
    cid = lax.axis_index("c")
    sid = lax.axis_index("s")
    wid = cid * 16 + sid
    lrow = sid * ROWS_PT

    pltpu.sync_copy(src_h.at[wid], src_v)
    pltpu.sync_copy(dst_h.at[wid], dst_v)
    pltpu.sync_copy(zeros_h, row_v)
    pltpu.sync_copy(row_v, acc_s.at[pl.ds(lrow, ROWS_PT)])
    plsc.subcore_barrier()

    for d in range(DEPTH):
        pltpu.async_copy(table.at[src_v.at[d]], msg_v.at[d], sem_g)

    def chunk(j, carry):
        b = lax.rem(j, NBUF)
        pltpu.make_async_copy(table.at[src_v.at[j]], msg_v.at[b], sem_g).wait()

        @pl.when(j + DEPTH < NCHUNK)
        def _():
            pltpu.async_copy(table.at[src_v.at[j + DEPTH]],
                             msg_v.at[lax.rem(j + DEPTH, NBUF)], sem_g)

        pltpu.async_copy(msg_v.at[b], acc_s.at[dst_v.at[j]], sem_s, add=True)

        @pl.when(j > 0)
        def _():
            pltpu.make_async_copy(
                msg_v.at[b], acc_s.at[dst_v.at[j]], sem_s).wait()

        return carry

    lax.fori_loop(0, NCHUNK, chunk, 0)
    pltpu.make_async_copy(msg_v.at[0], acc_s.at[dst_v.at[0]], sem_s).wait()
    plsc.subcore_barrier()

    pltpu.sync_copy(acc_s.at[pl.ds(lrow, ROWS_PT)], row_v)
    pltpu.sync_copy(row_v, acc_out.at[pl.ds(cid * NPAD + lrow, ROWS_PT)])


@jax.jit
def _sc_round(table, src_all, dst_all, zeros_rows):
    mesh = plsc.VectorSubcoreMesh(core_axis_name="c", subcore_axis_name="s")
    return pl.kernel(
        _sc_round_body,
        out_type=jax.ShapeDtypeStruct((2 * NPAD, 16), jnp.float32),
        mesh=mesh,
        scratch_types=[
            pltpu.VMEM((NCHUNK, CHUNK), jnp.int32),
            pltpu.VMEM((NCHUNK, CHUNK), jnp.int32),
            pltpu.VMEM((NBUF, CHUNK, 16), jnp.float32),
            pltpu.VMEM((ROWS_PT, 16), jnp.float32),
            pltpu.VMEM_SHARED((NPAD, 16), jnp.float32),
            pltpu.SemaphoreType.DMA,
            pltpu.SemaphoreType.DMA,
        ],
        compiler_params=pltpu.CompilerParams(use_tc_tiling_on_sc=False),
    )(table, src_all, dst_all, zeros_rows)


def _pre_body(px, lx, wrp, wsp, bp, wrl, wsl, bl, m1p, d1p, m1l, d1l):
    xp = px[...]
    xl = lx[...]
    m1p[...] = jnp.dot(xp, wrp[...], preferred_element_type=jnp.float32)
    d1p[...] = jnp.dot(xp, wsp[...], preferred_element_type=jnp.float32) + bp[...]
    m1l[...] = jnp.dot(xl, wrl[...], preferred_element_type=jnp.float32)
    d1l[...] = jnp.dot(xl, wsl[...], preferred_element_type=jnp.float32) + bl[...]


def _mid_body(a, d, o):
    o[...] = jnp.maximum(a[...] + d[...], 0.0)


def _head_body(a2p, h1p, a2l, h1l, act,
               wrp2, wsp2, bp2, wrl2, wsl2, bl2,
               pin_w, pin_b, ph_w, ph_b, po_w, po_b, out):
    p2 = (jnp.dot(a2p[...], wrp2[...], preferred_element_type=jnp.float32)
          + jnp.dot(h1p[...], wsp2[...], preferred_element_type=jnp.float32)
          + bp2[...])
    l2 = (jnp.dot(a2l[...], wrl2[...], preferred_element_type=jnp.float32)
          + jnp.dot(h1l[...], wsl2[...], preferred_element_type=jnp.float32)
          + bl2[...])
    mol = jnp.concatenate([p2, l2], axis=1)
    fp = jnp.maximum(
        jnp.dot(mol, pin_w[...], preferred_element_type=jnp.float32) + pin_b[...],
        0.0)
    pol = (jnp.dot(jnp.concatenate([fp, act[...]], axis=1), ph_w[...],
                   preferred_element_type=jnp.float32) + ph_b[...])
    out[...] = (jnp.dot(jnp.maximum(pol, 0.0), po_w[...],
                        preferred_element_type=jnp.float32) + po_b[...])


def _row_spec(block, width):
    return pl.BlockSpec((block, width), lambda i: (i, 0))


def _full_spec():
    return pl.BlockSpec()


def _pad_edges(src, dst, src_off, pad_src):
    src = jnp.concatenate(
        [src + src_off, jnp.full((EPAD - E,), pad_src, dtype=jnp.int32)])
    dst = jnp.concatenate([dst, jnp.zeros((EPAD - E,), dtype=jnp.int32)])
    return src, dst


def kernel(protein_x, ligand_x, action,
           pr_in_Wr, pr_in_Ws, pr_in_b, pr_out_Wr, pr_out_Ws, pr_out_b,
           lg_in_Wr, lg_in_Ws, lg_in_b, lg_out_Wr, lg_out_Ws, lg_out_b,
           pin_W, pin_b, ph_W, ph_b, po_W, po_b,
           protein_edge_index, ligand_edge_index):
    f32 = jnp.float32

    sp, dp = _pad_edges(protein_edge_index[0], protein_edge_index[1], 0, N)
    sl, dl = _pad_edges(ligand_edge_index[0], ligand_edge_index[1], NPAD, NPAD + N)
    src_all = jnp.concatenate([sp, sl]).reshape(32, NCHUNK, CHUNK)
    dst_all = jnp.concatenate([dp, dl]).reshape(32, NCHUNK, CHUNK)
    zeros_rows = jnp.zeros((ROWS_PT, 16), f32)

    blk = 2000
    grid = (N // blk,)
    m1p, d1p, m1l, d1l = pl.pallas_call(
        _pre_body,
        grid=grid,
        in_specs=[_row_spec(blk, D), _row_spec(blk, D)] + [_full_spec()] * 6,
        out_specs=[_row_spec(blk, 16)] * 4,
        out_shape=[jax.ShapeDtypeStruct((N, 16), f32)] * 4,
    )(protein_x, ligand_x,
      pr_in_Wr.T, pr_in_Ws.T, pr_in_b.reshape(1, 16),
      lg_in_Wr.T, lg_in_Ws.T, lg_in_b.reshape(1, 16))

    table1 = jnp.zeros((2 * NPAD, 16), f32)
    table1 = lax.dynamic_update_slice(table1, m1p, (0, 0))
    table1 = lax.dynamic_update_slice(table1, m1l, (NPAD, 0))
    d1 = jnp.zeros((2 * NPAD, 16), f32)
    d1 = lax.dynamic_update_slice(d1, d1p, (0, 0))
    d1 = lax.dynamic_update_slice(d1, d1l, (NPAD, 0))

    acc1 = _sc_round(table1, src_all, dst_all, zeros_rows)

    mblk = 2048
    table2 = pl.pallas_call(
        _mid_body,
        grid=(2 * NPAD // mblk,),
        in_specs=[_row_spec(mblk, 16)] * 2,
        out_specs=_row_spec(mblk, 16),
        out_shape=jax.ShapeDtypeStruct((2 * NPAD, 16), f32),
    )(acc1, d1)

    acc2 = _sc_round(table2, src_all, dst_all, zeros_rows)

    h1p = lax.dynamic_slice(table2, (0, 0), (N, 16))
    h1l = lax.dynamic_slice(table2, (NPAD, 0), (N, 16))
    a2p = lax.dynamic_slice(acc2, (0, 0), (N, 16))
    a2l = lax.dynamic_slice(acc2, (NPAD, 0), (N, 16))

    out = pl.pallas_call(
        _head_body,
        grid=grid,
        in_specs=[_row_spec(blk, 16)] * 4 + [_row_spec(blk, A)]
                 + [_full_spec()] * 12,
        out_specs=_row_spec(blk, 1),
        out_shape=jax.ShapeDtypeStruct((N, 1), f32),
    )(a2p, h1p, a2l, h1l, action,
      pr_out_Wr.T, pr_out_Ws.T, pr_out_b.reshape(1, 50),
      lg_out_Wr.T, lg_out_Ws.T, lg_out_b.reshape(1, 50),
      pin_W.T, pin_b.reshape(1, 60), ph_W.T, ph_b.reshape(1, 10),
      po_W.T, po_b.reshape(1, 1))
    return out


# speedup vs baseline: 16.3744x; 1.0512x over previous
import functools

import jax
import jax.numpy as jnp
from jax import lax
from jax.experimental import pallas as pl
from jax.experimental.pallas import tpu as pltpu
from jax.experimental.pallas import tpu_sc as plsc

N = 10000
E = 320000
D = 128
A = 32

NPAD = 10240
ROWS_PT = NPAD // 16
CHUNK = 128
NCHUNK = 157
EPT = NCHUNK * CHUNK
EPAD = 16 * EPT


DEPTH = 6
NBUF = 8


def _mp_round(table_of, src_v, dst_v, msg_v, acc_s, sem_g, sem_s):
    for d in range(DEPTH):
        pltpu.async_copy(table_of(src_v.at[d]), msg_v.at[d], sem_g)

    def chunk(j, carry):
        b = lax.rem(j, NBUF)
        pltpu.make_async_copy(table_of(src_v.at[j]), msg_v.at[b], sem_g).wait()

        @pl.when(j + DEPTH < NCHUNK)
        def _():
            pltpu.async_copy(table_of(src_v.at[j + DEPTH]),
                             msg_v.at[lax.rem(j + DEPTH, NBUF)], sem_g)

        pltpu.async_copy(msg_v.at[b], acc_s.at[dst_v.at[j]], sem_s, add=True)

        @pl.when(j > 0)
        def _():
            pltpu.make_async_copy(
                msg_v.at[b], acc_s.at[dst_v.at[j]], sem_s).wait()

        return carry

    lax.fori_loop(0, NCHUNK, chunk, 0)
    pltpu.make_async_copy(msg_v.at[0], acc_s.at[dst_v.at[0]], sem_s).wait()


def _sc_body(table1, d1h, src_h, dst_h, zeros_h, h1_out, acc2_out,
             src_v, dst_v, msg_v, row_v, rowd_v, acc_s, sem_g, sem_s):
    cid = lax.axis_index("c")
    sid = lax.axis_index("s")
    wid = cid * 16 + sid
    lrow = sid * ROWS_PT
    rows = pl.ds(lrow, ROWS_PT)

    pltpu.sync_copy(src_h.at[wid], src_v)
    pltpu.sync_copy(dst_h.at[wid], dst_v)
    pltpu.sync_copy(zeros_h, row_v)
    pltpu.sync_copy(row_v, acc_s.at[rows])
    plsc.subcore_barrier()

    _mp_round(lambda idx: table1.at[cid].at[idx],
              src_v, dst_v, msg_v, acc_s, sem_g, sem_s)
    plsc.subcore_barrier()

    pltpu.sync_copy(acc_s.at[rows], row_v)
    pltpu.sync_copy(d1h.at[cid].at[rows], rowd_v)

    def act(i, carry):
        row_v[i] = jnp.maximum(row_v[i] + rowd_v[i], 0.0)
        return carry

    lax.fori_loop(0, ROWS_PT, act, 0)
    pltpu.sync_copy(row_v, h1_out.at[cid].at[rows])
    pltpu.sync_copy(zeros_h, rowd_v)
    pltpu.sync_copy(rowd_v, acc_s.at[rows])
    plsc.subcore_barrier()

    _mp_round(lambda idx: h1_out.at[cid].at[idx],
              src_v, dst_v, msg_v, acc_s, sem_g, sem_s)
    plsc.subcore_barrier()

    pltpu.sync_copy(acc_s.at[rows], row_v)
    pltpu.sync_copy(row_v, acc2_out.at[cid].at[rows])


@jax.jit
def _sc_mp(table1, d1, src_all, dst_all, zeros_rows):
    mesh = plsc.VectorSubcoreMesh(core_axis_name="c", subcore_axis_name="s")
    return pl.kernel(
        _sc_body,
        out_type=(jax.ShapeDtypeStruct((2, NPAD, 16), jnp.float32),
                  jax.ShapeDtypeStruct((2, NPAD, 16), jnp.float32)),
        mesh=mesh,
        scratch_types=[
            pltpu.VMEM((NCHUNK, CHUNK), jnp.int32),
            pltpu.VMEM((NCHUNK, CHUNK), jnp.int32),
            pltpu.VMEM((NBUF, CHUNK, 16), jnp.float32),
            pltpu.VMEM((ROWS_PT, 16), jnp.float32),
            pltpu.VMEM((ROWS_PT, 16), jnp.float32),
            pltpu.VMEM_SHARED((NPAD, 16), jnp.float32),
            pltpu.SemaphoreType.DMA,
            pltpu.SemaphoreType.DMA,
        ],
        compiler_params=pltpu.CompilerParams(use_tc_tiling_on_sc=False),
    )(table1, d1, src_all, dst_all, zeros_rows)


def _pre_body(px, lx, wrp, wsp, bp, wrl, wsl, bl, m1p, d1p, m1l, d1l):
    xp = px[...]
    xl = lx[...]
    m1p[...] = jnp.dot(xp, wrp[...], preferred_element_type=jnp.float32)
    d1p[...] = jnp.dot(xp, wsp[...], preferred_element_type=jnp.float32) + bp[...]
    m1l[...] = jnp.dot(xl, wrl[...], preferred_element_type=jnp.float32)
    d1l[...] = jnp.dot(xl, wsl[...], preferred_element_type=jnp.float32) + bl[...]


def _head_body(a2p, h1p, a2l, h1l, act,
               wrp2, wsp2, bp2, wrl2, wsl2, bl2,
               pin_w, pin_b, ph_w, ph_b, po_w, po_b, out):
    p2 = (jnp.dot(a2p[...], wrp2[...], preferred_element_type=jnp.float32)
          + jnp.dot(h1p[...], wsp2[...], preferred_element_type=jnp.float32)
          + bp2[...])
    l2 = (jnp.dot(a2l[...], wrl2[...], preferred_element_type=jnp.float32)
          + jnp.dot(h1l[...], wsl2[...], preferred_element_type=jnp.float32)
          + bl2[...])
    mol = jnp.concatenate([p2, l2], axis=1)
    fp = jnp.maximum(
        jnp.dot(mol, pin_w[...], preferred_element_type=jnp.float32) + pin_b[...],
        0.0)
    pol = (jnp.dot(jnp.concatenate([fp, act[...]], axis=1), ph_w[...],
                   preferred_element_type=jnp.float32) + ph_b[...])
    out[...] = (jnp.dot(jnp.maximum(pol, 0.0), po_w[...],
                        preferred_element_type=jnp.float32) + po_b[...])


def _row_spec(block, width):
    return pl.BlockSpec((block, width), lambda i: (i, 0))


def _full_spec():
    return pl.BlockSpec()


def _pad_edges(src, dst, src_off, pad_src):
    src = jnp.concatenate(
        [src + src_off, jnp.full((EPAD - E,), pad_src, dtype=jnp.int32)])
    dst = jnp.concatenate([dst, jnp.zeros((EPAD - E,), dtype=jnp.int32)])
    return src, dst


def kernel(protein_x, ligand_x, action,
           pr_in_Wr, pr_in_Ws, pr_in_b, pr_out_Wr, pr_out_Ws, pr_out_b,
           lg_in_Wr, lg_in_Ws, lg_in_b, lg_out_Wr, lg_out_Ws, lg_out_b,
           pin_W, pin_b, ph_W, ph_b, po_W, po_b,
           protein_edge_index, ligand_edge_index):
    f32 = jnp.float32

    sp, dp = _pad_edges(protein_edge_index[0], protein_edge_index[1], 0, N)
    sl, dl = _pad_edges(ligand_edge_index[0], ligand_edge_index[1], 0, N)
    src_all = jnp.concatenate([sp, sl]).reshape(32, NCHUNK, CHUNK)
    dst_all = jnp.concatenate([dp, dl]).reshape(32, NCHUNK, CHUNK)
    zeros_rows = jnp.zeros((ROWS_PT, 16), f32)

    blk = 2000
    grid = (N // blk,)
    m1p, d1p, m1l, d1l = pl.pallas_call(
        _pre_body,
        grid=grid,
        in_specs=[_row_spec(blk, D), _row_spec(blk, D)] + [_full_spec()] * 6,
        out_specs=[_row_spec(blk, 16)] * 4,
        out_shape=[jax.ShapeDtypeStruct((N, 16), f32)] * 4,
    )(protein_x, ligand_x,
      pr_in_Wr.T, pr_in_Ws.T, pr_in_b.reshape(1, 16),
      lg_in_Wr.T, lg_in_Ws.T, lg_in_b.reshape(1, 16))

    table1 = jnp.zeros((2, NPAD, 16), f32)
    table1 = lax.dynamic_update_slice(table1, m1p[None], (0, 0, 0))
    table1 = lax.dynamic_update_slice(table1, m1l[None], (1, 0, 0))
    d1 = jnp.zeros((2, NPAD, 16), f32)
    d1 = lax.dynamic_update_slice(d1, d1p[None], (0, 0, 0))
    d1 = lax.dynamic_update_slice(d1, d1l[None], (1, 0, 0))

    h1, acc2 = _sc_mp(table1, d1, src_all, dst_all, zeros_rows)

    h1p = lax.dynamic_slice(h1, (0, 0, 0), (1, N, 16))[0]
    h1l = lax.dynamic_slice(h1, (1, 0, 0), (1, N, 16))[0]
    a2p = lax.dynamic_slice(acc2, (0, 0, 0), (1, N, 16))[0]
    a2l = lax.dynamic_slice(acc2, (1, 0, 0), (1, N, 16))[0]

    out = pl.pallas_call(
        _head_body,
        grid=grid,
        in_specs=[_row_spec(blk, 16)] * 4 + [_row_spec(blk, A)]
                 + [_full_spec()] * 12,
        out_specs=_row_spec(blk, 1),
        out_shape=jax.ShapeDtypeStruct((N, 1), f32),
    )(a2p, h1p, a2l, h1l, action,
      pr_out_Wr.T, pr_out_Ws.T, pr_out_b.reshape(1, 50),
      lg_out_Wr.T, lg_out_Ws.T, lg_out_b.reshape(1, 50),
      pin_W.T, pin_b.reshape(1, 60), ph_W.T, ph_b.reshape(1, 10),
      po_W.T, po_b.reshape(1, 1))
    return out


# speedup vs baseline: 20.4645x; 1.2498x over previous
import jax
import jax.numpy as jnp
from jax import lax
from jax.experimental import pallas as pl
from jax.experimental.pallas import tpu as pltpu
from jax.experimental.pallas import tpu_sc as plsc

N = 10000
E = 320000
D = 128
A = 32

ROWS_PT = N // 16
CHUNK = 125
NCHUNK = 160
EROW = E // CHUNK

DEPTH = 6
NBUF = 8


def _mp_round(table_of, src_v, dst_v, msg_v, acc_s, sem_g, sem_s):
    for d in range(DEPTH):
        pltpu.async_copy(table_of(src_v.at[d]), msg_v.at[d], sem_g)

    def chunk(j, carry):
        b = lax.rem(j, NBUF)
        pltpu.make_async_copy(table_of(src_v.at[j]), msg_v.at[b], sem_g).wait()

        @pl.when(j + DEPTH < NCHUNK)
        def _():
            pltpu.async_copy(table_of(src_v.at[j + DEPTH]),
                             msg_v.at[lax.rem(j + DEPTH, NBUF)], sem_g)

        pltpu.async_copy(msg_v.at[b], acc_s.at[dst_v.at[j]], sem_s, add=True)

        @pl.when(j > 0)
        def _():
            pltpu.make_async_copy(
                msg_v.at[b], acc_s.at[dst_v.at[j]], sem_s).wait()

        return carry

    lax.fori_loop(0, NCHUNK, chunk, 0)
    pltpu.make_async_copy(msg_v.at[0], acc_s.at[dst_v.at[0]], sem_s).wait()


def _sc_body(table1, d1h, pei, lei, zeros_h, h1_out, acc2_out,
             src_v, dst_v, msg_v, row_v, rowd_v, acc_s, sem_g, sem_s):
    cid = lax.axis_index("c")
    sid = lax.axis_index("s")
    lrow = sid * ROWS_PT
    rows = pl.ds(lrow, ROWS_PT)
    echunks = pl.ds(sid * NCHUNK, NCHUNK)

    @pl.when(cid == 0)
    def _():
        pltpu.sync_copy(pei.at[0].at[echunks], src_v)
        pltpu.sync_copy(pei.at[1].at[echunks], dst_v)

    @pl.when(cid == 1)
    def _():
        pltpu.sync_copy(lei.at[0].at[echunks], src_v)
        pltpu.sync_copy(lei.at[1].at[echunks], dst_v)

    pltpu.sync_copy(zeros_h, row_v)
    pltpu.sync_copy(row_v, acc_s.at[rows])
    plsc.subcore_barrier()

    _mp_round(lambda idx: table1.at[cid].at[idx],
              src_v, dst_v, msg_v, acc_s, sem_g, sem_s)
    plsc.subcore_barrier()

    pltpu.sync_copy(acc_s.at[rows], row_v)
    pltpu.sync_copy(d1h.at[cid].at[rows], rowd_v)

    def act(i, carry):
        row_v[i] = jnp.maximum(row_v[i] + rowd_v[i], 0.0)
        return carry

    lax.fori_loop(0, ROWS_PT, act, 0)
    pltpu.sync_copy(row_v, h1_out.at[cid].at[rows])
    pltpu.sync_copy(zeros_h, rowd_v)
    pltpu.sync_copy(rowd_v, acc_s.at[rows])
    plsc.subcore_barrier()

    _mp_round(lambda idx: h1_out.at[cid].at[idx],
              src_v, dst_v, msg_v, acc_s, sem_g, sem_s)
    plsc.subcore_barrier()

    pltpu.sync_copy(acc_s.at[rows], row_v)
    pltpu.sync_copy(row_v, acc2_out.at[cid].at[rows])


@jax.jit
def _sc_mp(table1, d1, pei, lei, zeros_rows):
    mesh = plsc.VectorSubcoreMesh(core_axis_name="c", subcore_axis_name="s")
    return pl.kernel(
        _sc_body,
        out_type=(jax.ShapeDtypeStruct((2, N, 16), jnp.float32),
                  jax.ShapeDtypeStruct((2, N, 16), jnp.float32)),
        mesh=mesh,
        scratch_types=[
            pltpu.VMEM((NCHUNK, CHUNK), jnp.int32),
            pltpu.VMEM((NCHUNK, CHUNK), jnp.int32),
            pltpu.VMEM((NBUF, CHUNK, 16), jnp.float32),
            pltpu.VMEM((ROWS_PT, 16), jnp.float32),
            pltpu.VMEM((ROWS_PT, 16), jnp.float32),
            pltpu.VMEM_SHARED((N, 16), jnp.float32),
            pltpu.SemaphoreType.DMA,
            pltpu.SemaphoreType.DMA,
        ],
        compiler_params=pltpu.CompilerParams(use_tc_tiling_on_sc=False),
    )(table1, d1, pei, lei, zeros_rows)


BLK = 5000


def _pre_body(px, lx, wrp, wsp, bp, wrl, wsl, bl, m1, d1):
    xp = px[...]
    xl = lx[...]
    m1[0] = jnp.dot(xp, wrp[...], preferred_element_type=jnp.float32)
    m1[1] = jnp.dot(xl, wrl[...], preferred_element_type=jnp.float32)
    d1[0] = jnp.dot(xp, wsp[...], preferred_element_type=jnp.float32) + bp[...]
    d1[1] = jnp.dot(xl, wsl[...], preferred_element_type=jnp.float32) + bl[...]


def _head_body(a2p, h1p, a2l, h1l, act,
               wrp2, wsp2, bp2, wrl2, wsl2, bl2,
               pin_w, pin_b, ph_w, ph_b, po_w, po_b, out):
    p2 = (jnp.dot(a2p[0], wrp2[...], preferred_element_type=jnp.float32)
          + jnp.dot(h1p[0], wsp2[...], preferred_element_type=jnp.float32)
          + bp2[...])
    l2 = (jnp.dot(a2l[0], wrl2[...], preferred_element_type=jnp.float32)
          + jnp.dot(h1l[0], wsl2[...], preferred_element_type=jnp.float32)
          + bl2[...])
    mol = jnp.concatenate([p2, l2], axis=1)
    fp = jnp.maximum(
        jnp.dot(mol, pin_w[...], preferred_element_type=jnp.float32) + pin_b[...],
        0.0)
    pol = (jnp.dot(jnp.concatenate([fp, act[...]], axis=1), ph_w[...],
                   preferred_element_type=jnp.float32) + ph_b[...])
    out[...] = (jnp.dot(jnp.maximum(pol, 0.0), po_w[...],
                        preferred_element_type=jnp.float32) + po_b[...])


def _row_spec(block, width):
    return pl.BlockSpec((block, width), lambda i: (i, 0))


def _plane_spec(plane):
    return pl.BlockSpec((1, BLK, 16), lambda i, p=plane: (p, i, 0))


def _full_spec():
    return pl.BlockSpec()


def kernel(protein_x, ligand_x, action,
           pr_in_Wr, pr_in_Ws, pr_in_b, pr_out_Wr, pr_out_Ws, pr_out_b,
           lg_in_Wr, lg_in_Ws, lg_in_b, lg_out_Wr, lg_out_Ws, lg_out_b,
           pin_W, pin_b, ph_W, ph_b, po_W, po_b,
           protein_edge_index, ligand_edge_index):
    f32 = jnp.float32
    grid = (N // BLK,)

    pei = protein_edge_index.reshape(2, EROW, CHUNK)
    lei = ligand_edge_index.reshape(2, EROW, CHUNK)
    zeros_rows = jnp.zeros((ROWS_PT, 16), f32)

    table1, d1 = pl.pallas_call(
        _pre_body,
        grid=grid,
        in_specs=[_row_spec(BLK, D), _row_spec(BLK, D)] + [_full_spec()] * 6,
        out_specs=[pl.BlockSpec((2, BLK, 16), lambda i: (0, i, 0))] * 2,
        out_shape=[jax.ShapeDtypeStruct((2, N, 16), f32)] * 2,
    )(protein_x, ligand_x,
      pr_in_Wr.T, pr_in_Ws.T, pr_in_b.reshape(1, 16),
      lg_in_Wr.T, lg_in_Ws.T, lg_in_b.reshape(1, 16))

    h1, acc2 = _sc_mp(table1, d1, pei, lei, zeros_rows)

    out = pl.pallas_call(
        _head_body,
        grid=grid,
        in_specs=[_plane_spec(0), _plane_spec(0), _plane_spec(1),
                  _plane_spec(1), _row_spec(BLK, A)] + [_full_spec()] * 12,
        out_specs=_row_spec(BLK, 1),
        out_shape=jax.ShapeDtypeStruct((N, 1), f32),
    )(acc2, h1, acc2, h1, action,
      pr_out_Wr.T, pr_out_Ws.T, pr_out_b.reshape(1, 50),
      lg_out_Wr.T, lg_out_Ws.T, lg_out_b.reshape(1, 50),
      pin_W.T, pin_b.reshape(1, 60), ph_W.T, ph_b.reshape(1, 10),
      po_W.T, po_b.reshape(1, 1))
    return out
